# Initial kernel scaffold; baseline (speedup 1.0000x reference)
import jax
import jax.numpy as jnp
from jax.experimental import pallas as pl


def kernel(tokens, tokens_style, audio, audio_noizy, times, token_table, style_table, W, b):
    raise NotImplementedError("write your pallas kernel here")


# speedup vs baseline: 1.2024x; 1.2024x over previous
import functools

import jax
import jax.numpy as jnp
from jax import lax
from jax.experimental import pallas as pl
from jax.experimental.pallas import tpu as pltpu
from jax.experimental.pallas import tpu_sc as plsc

_B, _T, _E, _D = 16, 2048, 512, 512
_ROWS = _B * _T
_NW = 32
_RPW = _ROWS // _NW
_CHUNK = 32
_NCHUNK = _RPW // _CHUNK
_LANES = 16


def _tables_body(tok_ref, sty_ref, w_ref, b_ref, tt_ref, st_ref):
    w = w_ref[...]
    tt_ref[...] = (
        jnp.dot(tok_ref[...], w, preferred_element_type=jnp.float32) + b_ref[...]
    )
    st_ref[...] = jnp.dot(sty_ref[...], w, preferred_element_type=jnp.float32)


def _fused_tables(token_table, style_table, W, b):
    return pl.pallas_call(
        _tables_body,
        out_shape=[
            jax.ShapeDtypeStruct((1000, _D), jnp.float32),
            jax.ShapeDtypeStruct((256, _D), jnp.float32),
        ],
    )(token_table, style_table, W, b.reshape(1, _D))


def _sc_body(tok_hbm, sty_hbm, audio_hbm, noizy_hbm, times_hbm, tt_hbm, st_hbm,
             out_hbm, tokv, styv, accv, audv, ttv, stv, timesv, s1, s2, s3, s4):
    wid = lax.axis_index("s") * 2 + lax.axis_index("c")
    base0 = wid * _RPW
    b_idx = wid // 2

    pltpu.sync_copy(times_hbm.at[b_idx], timesv)
    tvec = timesv[...]

    def chunk(ci, carry):
        base = base0 + ci * _CHUNK
        pltpu.sync_copy(tok_hbm.at[pl.ds(base, _CHUNK)], tokv)
        pltpu.sync_copy(sty_hbm.at[pl.ds(base, _CHUNK)], styv)
        cp1 = pltpu.async_copy(tt_hbm.at[tokv], ttv, s1)
        cp2 = pltpu.async_copy(st_hbm.at[styv], stv, s2)
        cp3 = pltpu.async_copy(noizy_hbm.at[pl.ds(base, _CHUNK)], accv, s3)
        cp4 = pltpu.async_copy(audio_hbm.at[pl.ds(base, _CHUNK)], audv, s4)
        cp1.wait()
        cp2.wait()
        cp3.wait()
        cp4.wait()

        def row(r, carry2):
            for k in range(_D // _LANES):
                sl = pl.ds(k * _LANES, _LANES)
                accv[r, sl] = (
                    accv[r, sl] + ttv[r, sl] + stv[r, sl] + tvec * audv[r, sl]
                )
            return carry2

        lax.fori_loop(0, _CHUNK, row, 0)
        pltpu.sync_copy(accv, out_hbm.at[pl.ds(base, _CHUNK)])
        return carry

    lax.fori_loop(0, _NCHUNK, chunk, 0)


@functools.partial(jax.jit)
def kernel(tokens, tokens_style, audio, audio_noizy, times, token_table,
           style_table, W, b):
    tt, st = _fused_tables(token_table, style_table, W, b)

    mesh = plsc.VectorSubcoreMesh(core_axis_name="c", subcore_axis_name="s")
    sc = pl.kernel(
        _sc_body,
        out_type=jax.ShapeDtypeStruct((_ROWS, _D), jnp.float32),
        mesh=mesh,
        scratch_types=[
            pltpu.VMEM((_CHUNK,), jnp.int32),
            pltpu.VMEM((_CHUNK,), jnp.int32),
            pltpu.VMEM((_CHUNK, _D), jnp.float32),
            pltpu.VMEM((_CHUNK, _D), jnp.float32),
            pltpu.VMEM((_CHUNK, _D), jnp.float32),
            pltpu.VMEM((_CHUNK, _D), jnp.float32),
            pltpu.VMEM((_LANES,), jnp.float32),
            pltpu.SemaphoreType.DMA,
            pltpu.SemaphoreType.DMA,
            pltpu.SemaphoreType.DMA,
            pltpu.SemaphoreType.DMA,
        ],
    )
    out = sc(
        tokens.reshape(_ROWS).astype(jnp.int32),
        tokens_style.reshape(_ROWS).astype(jnp.int32),
        audio.reshape(_ROWS, _D),
        audio_noizy.reshape(_ROWS, _D),
        jnp.broadcast_to(times[:, None], (_B, _LANES)),
        tt,
        st,
    )
    return out.reshape(_B, _T, _D)


# speedup vs baseline: 1.8611x; 1.5479x over previous
import functools

import jax
import jax.numpy as jnp
from jax import lax
from jax.experimental import pallas as pl
from jax.experimental.pallas import tpu as pltpu
from jax.experimental.pallas import tpu_sc as plsc

_B, _T, _E, _D = 16, 2048, 512, 512
_ROWS = _B * _T
_NW = 32
_RPW = _ROWS // _NW
_R = 16
_NB = 2
_NCHUNK = _RPW // _R
_LANES = 16
_CHUNK_BYTES = _R * _D * 4


def _tables_body(tok_ref, sty_ref, w_ref, b_ref, tt_ref, st_ref):
    w = w_ref[...]
    tt_ref[...] = (
        jnp.dot(tok_ref[...], w, preferred_element_type=jnp.float32) + b_ref[...]
    )
    st_ref[...] = jnp.dot(sty_ref[...], w, preferred_element_type=jnp.float32)


def _fused_tables(token_table, style_table, W, b):
    return pl.pallas_call(
        _tables_body,
        out_shape=[
            jax.ShapeDtypeStruct((1000, _D), jnp.float32),
            jax.ShapeDtypeStruct((256, _D), jnp.float32),
        ],
    )(token_table, style_table, W, b.reshape(1, _D))


def _sc_body(tok_hbm, sty_hbm, audio_hbm, noizy_hbm, times_hbm, tt_hbm, st_hbm,
             out_hbm, tokidx, styidx, timesv, noizyb, audb, ttb, stb, outb,
             insems, outsems):
    wid = lax.axis_index("s") * 2 + lax.axis_index("c")
    base0 = wid * _RPW
    b_idx = wid // 2

    pltpu.sync_copy(tok_hbm.at[pl.ds(base0, _RPW)], tokidx)
    pltpu.sync_copy(sty_hbm.at[pl.ds(base0, _RPW)], styidx)
    pltpu.sync_copy(times_hbm.at[b_idx], timesv)
    tvec = timesv[...]

    def issue_in(ci, s):
        base = base0 + ci * _R
        lo = ci * _R
        pltpu.async_copy(tt_hbm.at[tokidx.at[pl.ds(lo, _R)]], ttb.at[s],
                         insems[s])
        pltpu.async_copy(st_hbm.at[styidx.at[pl.ds(lo, _R)]], stb.at[s],
                         insems[s])
        pltpu.async_copy(noizy_hbm.at[pl.ds(base, _R)], noizyb.at[s], insems[s])
        pltpu.async_copy(audio_hbm.at[pl.ds(base, _R)], audb.at[s], insems[s])

    def wait_in(ci, s):
        lo = ci * _R
        base = base0 + ci * _R
        pltpu.make_async_copy(tt_hbm.at[tokidx.at[pl.ds(lo, _R)]], ttb.at[s],
                              insems[s]).wait()
        pltpu.make_async_copy(st_hbm.at[styidx.at[pl.ds(lo, _R)]], stb.at[s],
                              insems[s]).wait()
        pltpu.make_async_copy(noizy_hbm.at[pl.ds(base, _R)], noizyb.at[s],
                              insems[s]).wait()
        pltpu.make_async_copy(audio_hbm.at[pl.ds(base, _R)], audb.at[s],
                              insems[s]).wait()

    def issue_out(ci, s):
        base = base0 + ci * _R
        pltpu.async_copy(outb.at[s], out_hbm.at[pl.ds(base, _R)], outsems[s])

    def wait_out(ci, s):
        base = base0 + ci * _R
        pltpu.make_async_copy(outb.at[s], out_hbm.at[pl.ds(base, _R)],
                              outsems[s]).wait()

    def compute(s):
        def row(r, c2):
            for k in range(_D // _LANES):
                sl = pl.ds(k * _LANES, _LANES)
                outb[s, r, sl] = (
                    noizyb[s, r, sl] + ttb[s, r, sl] + stb[s, r, sl]
                    + tvec * audb[s, r, sl]
                )
            return c2

        lax.fori_loop(0, _R, row, 0)

    issue_in(0, 0)

    @pl.loop(0, _NCHUNK, step=_NB)
    def outer(ci0):
        for s in range(_NB):
            ci = ci0 + s
            wait_in(ci, s)
            ns = (s + 1) % _NB

            @pl.when(ci + 1 < _NCHUNK)
            def _():
                issue_in(ci + 1, ns)

            @pl.when(ci >= _NB)
            def _():
                wait_out(ci - _NB, s)

            compute(s)
            issue_out(ci, s)

    for s in range(_NB):
        wait_out(_NCHUNK - _NB + s, s)


@functools.partial(jax.jit)
def kernel(tokens, tokens_style, audio, audio_noizy, times, token_table,
           style_table, W, b):
    tt, st = _fused_tables(token_table, style_table, W, b)

    mesh = plsc.VectorSubcoreMesh(core_axis_name="c", subcore_axis_name="s")
    sc = pl.kernel(
        _sc_body,
        out_type=jax.ShapeDtypeStruct((_ROWS, _D), jnp.float32),
        mesh=mesh,
        scratch_types=[
            pltpu.VMEM((_RPW,), jnp.int32),
            pltpu.VMEM((_RPW,), jnp.int32),
            pltpu.VMEM((_LANES,), jnp.float32),
            pltpu.VMEM((_NB, _R, _D), jnp.float32),
            pltpu.VMEM((_NB, _R, _D), jnp.float32),
            pltpu.VMEM((_NB, _R, _D), jnp.float32),
            pltpu.VMEM((_NB, _R, _D), jnp.float32),
            pltpu.VMEM((_NB, _R, _D), jnp.float32),
            [pltpu.SemaphoreType.DMA] * _NB,
            [pltpu.SemaphoreType.DMA] * _NB,
        ],
    )
    out = sc(
        tokens.reshape(_ROWS).astype(jnp.int32),
        tokens_style.reshape(_ROWS).astype(jnp.int32),
        audio.reshape(_ROWS, _D),
        audio_noizy.reshape(_ROWS, _D),
        jnp.broadcast_to(times[:, None], (_B, _LANES)),
        tt,
        st,
    )
    return out.reshape(_B, _T, _D)


# speedup vs baseline: 1.9050x; 1.0236x over previous
import functools

import jax
import jax.numpy as jnp
from jax import lax
from jax.experimental import pallas as pl
from jax.experimental.pallas import tpu as pltpu
from jax.experimental.pallas import tpu_sc as plsc

_B, _T, _E, _D = 16, 2048, 512, 512
_ROWS = _B * _T
_NW = 32
_RPW = _ROWS // _NW
_R = 16
_NB = 2
_NCHUNK = _RPW // _R
_LANES = 16
_NTOK = 1000


def _tables_body(tok_ref, sty_ref, w_ref, b_ref, ttst_ref):
    w = w_ref[...]
    ttst_ref[: _NTOK, :] = (
        jnp.dot(tok_ref[...], w, preferred_element_type=jnp.float32) + b_ref[...]
    )
    ttst_ref[_NTOK:, :] = jnp.dot(sty_ref[...], w,
                                  preferred_element_type=jnp.float32)


def _fused_tables(token_table, style_table, W, b):
    return pl.pallas_call(
        _tables_body,
        out_shape=jax.ShapeDtypeStruct((_NTOK + 256, _D), jnp.float32),
    )(token_table, style_table, W, b.reshape(1, _D))


def _sc_body(cidx_hbm, audio_hbm, noizy_hbm, times_hbm, ttst_hbm,
             out_hbm, cidx, timesv, noizyb, audb, gatb, outb, insems, outsems):
    wid = lax.axis_index("s") * 2 + lax.axis_index("c")
    base0 = wid * _RPW
    b_idx = wid // 2

    pltpu.sync_copy(cidx_hbm.at[pl.ds(base0 * 2, _RPW * 2)], cidx)
    pltpu.sync_copy(times_hbm.at[b_idx], timesv)
    tvec = timesv[...]

    def issue_in(ci, s):
        base = base0 + ci * _R
        pltpu.async_copy(ttst_hbm.at[cidx.at[pl.ds(ci * 2 * _R, 2 * _R)]],
                         gatb.at[s], insems[s])
        pltpu.async_copy(noizy_hbm.at[pl.ds(base, _R)], noizyb.at[s], insems[s])
        pltpu.async_copy(audio_hbm.at[pl.ds(base, _R)], audb.at[s], insems[s])

    def wait_in(ci, s):
        base = base0 + ci * _R
        pltpu.make_async_copy(ttst_hbm.at[cidx.at[pl.ds(ci * 2 * _R, 2 * _R)]],
                              gatb.at[s], insems[s]).wait()
        pltpu.make_async_copy(noizy_hbm.at[pl.ds(base, _R)], noizyb.at[s],
                              insems[s]).wait()
        pltpu.make_async_copy(audio_hbm.at[pl.ds(base, _R)], audb.at[s],
                              insems[s]).wait()

    def issue_out(ci, s):
        base = base0 + ci * _R
        pltpu.async_copy(outb.at[s], out_hbm.at[pl.ds(base, _R)], outsems[s])

    def wait_out(ci, s):
        base = base0 + ci * _R
        pltpu.make_async_copy(outb.at[s], out_hbm.at[pl.ds(base, _R)],
                              outsems[s]).wait()

    def compute(s):
        def row(r, c2):
            for k in range(_D // _LANES):
                sl = pl.ds(k * _LANES, _LANES)
                outb[s, r, sl] = (
                    noizyb[s, r, sl] + gatb[s, r, sl] + gatb[s, _R + r, sl]
                    + tvec * audb[s, r, sl]
                )
            return c2

        lax.fori_loop(0, _R, row, 0)

    for p in range(_NB - 1):
        issue_in(p, p)

    @pl.loop(0, _NCHUNK, step=_NB)
    def outer(ci0):
        for s in range(_NB):
            ci = ci0 + s
            wait_in(ci, s)
            ns = (s + _NB - 1) % _NB

            @pl.when(ci + _NB - 1 < _NCHUNK)
            def _():
                issue_in(ci + _NB - 1, ns)

            @pl.when(ci >= _NB)
            def _():
                wait_out(ci - _NB, s)

            compute(s)
            issue_out(ci, s)

    for ci in range(_NCHUNK - _NB, _NCHUNK):
        wait_out(ci, ci % _NB)


@functools.partial(jax.jit)
def kernel(tokens, tokens_style, audio, audio_noizy, times, token_table,
           style_table, W, b):
    ttst = _fused_tables(token_table, style_table, W, b)

    tok = tokens.reshape(_NW, _NCHUNK, _R).astype(jnp.int32)
    sty = tokens_style.reshape(_NW, _NCHUNK, _R).astype(jnp.int32) + _NTOK
    cidx = jnp.concatenate([tok, sty], axis=2).reshape(_ROWS * 2)

    mesh = plsc.VectorSubcoreMesh(core_axis_name="c", subcore_axis_name="s")
    sc = pl.kernel(
        _sc_body,
        out_type=jax.ShapeDtypeStruct((_ROWS, _D), jnp.float32),
        mesh=mesh,
        scratch_types=[
            pltpu.VMEM((_RPW * 2,), jnp.int32),
            pltpu.VMEM((_LANES,), jnp.float32),
            pltpu.VMEM((_NB, _R, _D), jnp.float32),
            pltpu.VMEM((_NB, _R, _D), jnp.float32),
            pltpu.VMEM((_NB, 2 * _R, _D), jnp.float32),
            pltpu.VMEM((_NB, _R, _D), jnp.float32),
            [pltpu.SemaphoreType.DMA] * _NB,
            [pltpu.SemaphoreType.DMA] * _NB,
        ],
    )
    out = sc(
        cidx,
        audio.reshape(_ROWS, _D),
        audio_noizy.reshape(_ROWS, _D),
        jnp.broadcast_to(times[:, None], (_B, _LANES)),
        ttst,
    )
    return out.reshape(_B, _T, _D)
